# baseline (speedup 1.0000x reference)
import functools

import jax
from jax import lax
import jax.numpy as jnp
from jax.experimental import pallas as pl
from jax.experimental.pallas import tpu as pltpu
from jax.experimental.pallas import tpu_sc as plsc

_BLOCK = 20
_D = 16
_H = 64
_BT = 256
_NC = 2
_NS = 16
_EW = 384


def _sc_gather(table_p, idx_flat):
    n = idx_flat.shape[0]
    nbatch = n // _BLOCK
    nw = _NC * _NS
    per_w = n // nw
    nch = per_w // 160
    mesh = plsc.VectorSubcoreMesh(core_axis_name="c", subcore_axis_name="s")

    @functools.partial(
        pl.kernel,
        mesh=mesh,
        out_type=jax.ShapeDtypeStruct((nbatch, _EW), jnp.float32),
        scratch_types=[
            pltpu.VMEM((160,), jnp.int32),
            pltpu.VMEM((160, 128), jnp.float32),
            pltpu.VMEM((8, _EW), jnp.float32),
            pltpu.VMEM_SHARED((1008, 128), jnp.float32),
            pltpu.SemaphoreType.DMA,
            pltpu.SemaphoreType.DMA,
        ],
    )
    def gather_kernel(tab_hbm, i_hbm, o_hbm, idx_v, rows_v, comp_v, tab_sh,
                      sem, sem2):
        wid = lax.axis_index("s") * _NC + lax.axis_index("c")
        base = wid * per_w
        row_base = base // _BLOCK

        @pl.when(lax.axis_index("s") == 0)
        def _():
            pltpu.sync_copy(tab_hbm, tab_sh)

        plsc.subcore_barrier()
        for r in range(8):
            for s in range((_EW - _BLOCK * _D) // _D):
                comp_v[r, pl.ds(_BLOCK * _D + _D * s, _D)] = jnp.zeros(
                    (_D,), jnp.float32)

        @pl.loop(0, nch)
        def _(c):
            off = base + c * 160
            pltpu.sync_copy(i_hbm.at[pl.ds(off, 160)], idx_v)
            cp1 = pltpu.async_copy(
                tab_sh.at[idx_v.at[pl.ds(0, 80)]], rows_v.at[pl.ds(0, 80)], sem)
            cp2 = pltpu.async_copy(
                tab_sh.at[idx_v.at[pl.ds(80, 80)]], rows_v.at[pl.ds(80, 80)],
                sem2)
            cp1.wait()
            cp2.wait()
            for i in range(160):
                comp_v[i // _BLOCK, pl.ds(_D * (i % _BLOCK), _D)] = (
                    rows_v[i, pl.ds(0, _D)])
            pltpu.sync_copy(
                comp_v,
                o_hbm.at[pl.ds(pl.multiple_of(row_base + c * 8, 8), 8)])

    return gather_kernel(table_p, idx_flat)


_TS = 2


def _mlp_body(e_ref, pad_ref, w1lo_ref, w1hi_ref, b1b_ref, w2t_ref, b2t_ref,
              out_ref):
    padterm = lax.dot_general(
        w1hi_ref[...], pad_ref[...], (((1,), (1,)), ((), ())),
        preferred_element_type=jnp.float32)
    e2 = e_ref[...].astype(jnp.bfloat16)
    ht = jnp.tanh(
        lax.dot_general(w1lo_ref[...], e2, (((1,), (1,)), ((), ())),
                        preferred_element_type=jnp.float32)
        + padterm + b1b_ref[...]
    )
    htb = ht.astype(jnp.bfloat16)
    for t in range(_BLOCK // _TS):
        o = jnp.dot(w2t_ref[...], htb[_H * t:_H * (t + 1), :],
                    preferred_element_type=jnp.float32) + b2t_ref[...]
        out_ref[t] = o


def _mlp_body2(e_ref, pad_ref, w1lo_ref, w1hi_ref, b1b_ref, w2t_ref, b2t_ref,
               y_ref, out_ref):
    del y_ref
    _mlp_body(e_ref, pad_ref, w1lo_ref, w1hi_ref, b1b_ref, w2t_ref, b2t_ref,
              out_ref)


def kernel(idx, table, W1, b1, W2, b2):
    batch, block = idx.shape
    d = table.shape[1]
    v = W2.shape[1]
    q1 = batch // 4
    q2 = batch - q1

    table_p = jnp.pad(table, ((0, 7), (0, 128 - d)))
    e_h1 = _sc_gather(table_p, idx[:q1].reshape(-1))
    e_h2 = _sc_gather(table_p, idx[q1:].reshape(-1))
    pad19 = jnp.tile(table[-1], block - 1).reshape(1, (block - 1) * d)
    w1r = W1.reshape(block, d, -1)[::-1].reshape(block * d, -1)
    w1big = jnp.concatenate(
        [jnp.pad(w1r, ((d * t, (block - 1) * d - d * t), (0, 0)))
         for t in range(block)], axis=1)
    w1lo_t = jnp.pad(
        w1big[(block - 1) * d:].T.astype(jnp.bfloat16),
        ((0, 0), (0, _EW - block * d)))
    w1hi_t = w1big[:(block - 1) * d].T
    b1big = jnp.tile(b1, block).reshape(block * _H, 1)

    w2t = W2.T.astype(jnp.bfloat16)
    b2t = b2.reshape(v, 1)
    nb1 = q1 // _BT

    hrows = block * _H // _TS
    trows = block // _TS
    common_specs = [
        pl.BlockSpec(pad19.shape, lambda i, j: (0, 0)),
        pl.BlockSpec((hrows, _EW), lambda i, j: (j, 0)),
        pl.BlockSpec((hrows, (block - 1) * d), lambda i, j: (j, 0)),
        pl.BlockSpec((hrows, 1), lambda i, j: (j, 0)),
        pl.BlockSpec((v, _H), lambda i, j: (0, 0)),
        pl.BlockSpec((v, 1), lambda i, j: (0, 0)),
    ]
    out_shape = jax.ShapeDtypeStruct((block, v, batch), jnp.float32)

    y1 = pl.pallas_call(
        _mlp_body,
        grid=(nb1, _TS),
        in_specs=[pl.BlockSpec((_BT, _EW), lambda i, j: (i, 0))]
        + common_specs,
        out_specs=pl.BlockSpec((trows, v, _BT), lambda i, j: (j, 0, i)),
        out_shape=out_shape,
    )(e_h1, pad19, w1lo_t, w1hi_t, b1big, w2t, b2t)

    out_t = pl.pallas_call(
        _mlp_body2,
        grid=(q2 // _BT, _TS),
        in_specs=[pl.BlockSpec((_BT, _EW), lambda i, j: (i, 0))]
        + common_specs
        + [pl.BlockSpec(memory_space=pl.ANY)],
        out_specs=pl.BlockSpec((trows, v, _BT),
                               lambda i, j: (j, 0, i + nb1)),
        out_shape=out_shape,
        input_output_aliases={7: 0},
    )(e_h2, pad19, w1lo_t, w1hi_t, b1big, w2t, b2t, y1)
    return jnp.transpose(out_t, (2, 0, 1))


# speedup vs baseline: 1.1107x; 1.1107x over previous
import functools

import jax
from jax import lax
import jax.numpy as jnp
from jax.experimental import pallas as pl
from jax.experimental.pallas import tpu as pltpu
from jax.experimental.pallas import tpu_sc as plsc

_BLOCK = 20
_D = 16
_H = 64
_BT = 256
_NC = 2
_NS = 16
_EW = 384


def _sc_gather(table_p, idx_flat):
    n = idx_flat.shape[0]
    nbatch = n // _BLOCK
    nw = _NC * _NS
    per_w = n // nw
    nch = per_w // 160
    mesh = plsc.VectorSubcoreMesh(core_axis_name="c", subcore_axis_name="s")

    @functools.partial(
        pl.kernel,
        mesh=mesh,
        out_type=jax.ShapeDtypeStruct((nbatch, _EW), jnp.float32),
        scratch_types=[
            pltpu.VMEM((160,), jnp.int32),
            pltpu.VMEM((160, 128), jnp.float32),
            pltpu.VMEM((8, _EW), jnp.float32),
            pltpu.VMEM_SHARED((1008, 128), jnp.float32),
            pltpu.SemaphoreType.DMA,
            pltpu.SemaphoreType.DMA,
        ],
    )
    def gather_kernel(tab_hbm, i_hbm, o_hbm, idx_v, rows_v, comp_v, tab_sh,
                      sem, sem2):
        wid = lax.axis_index("s") * _NC + lax.axis_index("c")
        base = wid * per_w
        row_base = base // _BLOCK

        @pl.when(lax.axis_index("s") == 0)
        def _():
            pltpu.sync_copy(tab_hbm, tab_sh)

        plsc.subcore_barrier()
        for r in range(8):
            for s in range((_EW - _BLOCK * _D) // _D):
                comp_v[r, pl.ds(_BLOCK * _D + _D * s, _D)] = jnp.zeros(
                    (_D,), jnp.float32)

        @pl.loop(0, nch)
        def _(c):
            off = base + c * 160
            pltpu.sync_copy(i_hbm.at[pl.ds(off, 160)], idx_v)
            cp1 = pltpu.async_copy(
                tab_sh.at[idx_v.at[pl.ds(0, 80)]], rows_v.at[pl.ds(0, 80)], sem)
            cp2 = pltpu.async_copy(
                tab_sh.at[idx_v.at[pl.ds(80, 80)]], rows_v.at[pl.ds(80, 80)],
                sem2)
            cp1.wait()
            cp2.wait()
            for i in range(160):
                comp_v[i // _BLOCK, pl.ds(_D * (i % _BLOCK), _D)] = (
                    rows_v[i, pl.ds(0, _D)])
            pltpu.sync_copy(
                comp_v,
                o_hbm.at[pl.ds(pl.multiple_of(row_base + c * 8, 8), 8)])

    return gather_kernel(table_p, idx_flat)


def _mlp_body(e_ref, pad_ref, w1lo_ref, w1hi_ref, b1b_ref, w2t_ref, b2t_ref,
              out_ref):
    padterm = lax.dot_general(
        w1hi_ref[...], pad_ref[...], (((1,), (1,)), ((), ())),
        preferred_element_type=jnp.float32)
    e2 = e_ref[...].astype(jnp.bfloat16)
    ht = jnp.tanh(
        lax.dot_general(w1lo_ref[...], e2, (((1,), (1,)), ((), ())),
                        preferred_element_type=jnp.float32)
        + padterm + b1b_ref[...]
    )
    htb = ht.astype(jnp.bfloat16)
    for t in range(_BLOCK):
        o = jnp.dot(w2t_ref[...], htb[_H * t:_H * (t + 1), :],
                    preferred_element_type=jnp.float32) + b2t_ref[...]
        out_ref[t] = o


def _mlp_body2(e_ref, pad_ref, w1lo_ref, w1hi_ref, b1b_ref, w2t_ref, b2t_ref,
               y_ref, out_ref):
    del y_ref
    _mlp_body(e_ref, pad_ref, w1lo_ref, w1hi_ref, b1b_ref, w2t_ref, b2t_ref,
              out_ref)


def kernel(idx, table, W1, b1, W2, b2):
    batch, block = idx.shape
    d = table.shape[1]
    v = W2.shape[1]
    q1 = batch // 4
    q2 = batch - q1

    table_p = jnp.pad(table, ((0, 7), (0, 128 - d)))
    e_h1 = _sc_gather(table_p, idx[:q1].reshape(-1))
    e_h2 = _sc_gather(table_p, idx[q1:].reshape(-1))
    pad19 = jnp.tile(table[-1], block - 1).reshape(1, (block - 1) * d)
    w1r = W1.reshape(block, d, -1)[::-1].reshape(block * d, -1)
    w1big = jnp.concatenate(
        [jnp.pad(w1r, ((d * t, (block - 1) * d - d * t), (0, 0)))
         for t in range(block)], axis=1)
    w1lo_t = jnp.pad(
        w1big[(block - 1) * d:].T.astype(jnp.bfloat16),
        ((0, 0), (0, _EW - block * d)))
    w1hi_t = w1big[:(block - 1) * d].T
    b1big = jnp.tile(b1, block).reshape(block * _H, 1)

    w2t = W2.T.astype(jnp.bfloat16)
    b2t = b2.reshape(v, 1)
    nb1 = q1 // _BT

    common_specs = [
        pl.BlockSpec(pad19.shape, lambda i: (0, 0)),
        pl.BlockSpec(w1lo_t.shape, lambda i: (0, 0)),
        pl.BlockSpec(w1hi_t.shape, lambda i: (0, 0)),
        pl.BlockSpec(b1big.shape, lambda i: (0, 0)),
        pl.BlockSpec((v, _H), lambda i: (0, 0)),
        pl.BlockSpec((v, 1), lambda i: (0, 0)),
    ]
    out_shape = jax.ShapeDtypeStruct((block, v, batch), jnp.float32)

    y1 = pl.pallas_call(
        _mlp_body,
        grid=(nb1,),
        in_specs=[pl.BlockSpec((_BT, _EW), lambda i: (i, 0))]
        + common_specs,
        out_specs=pl.BlockSpec((block, v, _BT), lambda i: (0, 0, i)),
        out_shape=out_shape,
    )(e_h1, pad19, w1lo_t, w1hi_t, b1big, w2t, b2t)

    out_t = pl.pallas_call(
        _mlp_body2,
        grid=(q2 // _BT,),
        in_specs=[pl.BlockSpec((_BT, _EW), lambda i: (i, 0))]
        + common_specs
        + [pl.BlockSpec(memory_space=pl.ANY)],
        out_specs=pl.BlockSpec((block, v, _BT),
                               lambda i: (0, 0, i + nb1)),
        out_shape=out_shape,
        input_output_aliases={7: 0},
    )(e_h2, pad19, w1lo_t, w1hi_t, b1big, w2t, b2t, y1)
    return jnp.transpose(out_t, (2, 0, 1))
